# Initial kernel scaffold; baseline (speedup 1.0000x reference)
import jax
import jax.numpy as jnp
from jax.experimental import pallas as pl


def kernel(x, atom_emb_weight):
    raise NotImplementedError("write your pallas kernel here")


# speedup vs baseline: 5.0660x; 5.0660x over previous
import functools

import jax
import jax.numpy as jnp
from jax import lax
from jax.experimental import pallas as pl
from jax.experimental.pallas import tpu as pltpu
from jax.experimental.pallas import tpu_sc as plsc

EMB = 64
NC, NS = 2, 16
NW = NC * NS

K = 128
CHUNKS = 4
BLOCK = K * CHUNKS


@functools.lru_cache(maxsize=None)
def _make_kernel(B, V):
    xrows_per_blk = BLOCK // K
    blocks_per_w = B // (BLOCK * NW)
    assert blocks_per_w * BLOCK * NW == B and blocks_per_w % 2 == 0

    mesh = plsc.VectorSubcoreMesh(core_axis_name="c", subcore_axis_name="s")

    @functools.partial(
        pl.kernel,
        mesh=mesh,
        out_type=jax.ShapeDtypeStruct((B, EMB), jnp.float32),
        compiler_params=pltpu.CompilerParams(use_tc_tiling_on_sc=False),
        scratch_types=[
            pltpu.VMEM((2, CHUNKS, K), jnp.int32),
            pltpu.VMEM((2, BLOCK, EMB), jnp.float32),
            pltpu.SemaphoreType.DMA,
            pltpu.SemaphoreType.DMA,
        ],
    )
    def k(x_hbm, table_hbm, out_hbm, idx_v, rows_v, sem0, sem1):
        c = lax.axis_index("c")
        s = lax.axis_index("s")
        wid = s * NC + c
        xrow0 = wid * blocks_per_w * xrows_per_blk
        row0 = wid * blocks_per_w * BLOCK
        sems = (sem0, sem1)

        def load_idx(slot, blk):
            off = xrow0 + blk * xrows_per_blk
            pltpu.sync_copy(x_hbm.at[pl.ds(off, xrows_per_blk)], idx_v.at[slot])

        def fire(slot, sem):
            for j in range(CHUNKS):
                pltpu.async_copy(
                    table_hbm.at[idx_v.at[slot, j]],
                    rows_v.at[slot, pl.ds(j * K, K)],
                    sem,
                )

        def drain(slot, sem):
            for j in range(CHUNKS):
                pltpu.make_async_copy(
                    table_hbm.at[idx_v.at[slot, j]],
                    rows_v.at[slot, pl.ds(j * K, K)],
                    sem,
                ).wait()

        def store(slot, blk):
            off = row0 + blk * BLOCK
            pltpu.sync_copy(rows_v.at[slot], out_hbm.at[pl.ds(off, BLOCK)])

        load_idx(0, 0)
        fire(0, sem0)

        def body(i, carry):
            for phase in range(2):
                blk = i * 2 + phase
                slot = phase
                nslot = 1 - phase
                nblk = blk + 1

                @pl.when(nblk < blocks_per_w)
                def _():
                    load_idx(nslot, nblk)
                    fire(nslot, sems[nslot])

                drain(slot, sems[slot])
                store(slot, blk)
            return carry

        lax.fori_loop(0, blocks_per_w // 2, body, 0)

    return k


def kernel(x, atom_emb_weight):
    B = x.shape[0] * x.shape[1]
    x2 = x.reshape(B // K, K).astype(jnp.int32)
    out = _make_kernel(B, atom_emb_weight.shape[0])(x2, atom_emb_weight)
    return out.reshape(x.shape + (EMB,))


# speedup vs baseline: 5.1678x; 1.0201x over previous
import functools

import jax
import jax.numpy as jnp
from jax import lax
from jax.experimental import pallas as pl
from jax.experimental.pallas import tpu as pltpu
from jax.experimental.pallas import tpu_sc as plsc

EMB = 64
NC, NS = 2, 16
NW = NC * NS

K = 128
CHUNKS = 4
BLOCK = K * CHUNKS


@functools.lru_cache(maxsize=None)
def _make_kernel(B, V):
    xrows_per_blk = BLOCK // K
    nblk = B // (BLOCK * NW)
    assert nblk * BLOCK * NW == B and nblk % 4 == 0

    mesh = plsc.VectorSubcoreMesh(core_axis_name="c", subcore_axis_name="s")

    @functools.partial(
        pl.kernel,
        mesh=mesh,
        out_type=jax.ShapeDtypeStruct((B, EMB), jnp.float32),
        compiler_params=pltpu.CompilerParams(use_tc_tiling_on_sc=False),
        scratch_types=[
            pltpu.VMEM((4, CHUNKS, K), jnp.int32),
            pltpu.VMEM((2, BLOCK, EMB), jnp.float32),
            pltpu.SemaphoreType.DMA,
            pltpu.SemaphoreType.DMA,
            pltpu.SemaphoreType.DMA,
            pltpu.SemaphoreType.DMA,
            pltpu.SemaphoreType.DMA,
            pltpu.SemaphoreType.DMA,
            pltpu.SemaphoreType.DMA,
            pltpu.SemaphoreType.DMA,
        ],
    )
    def k(x_hbm, table_hbm, out_hbm, idx_v, rows_v,
          g0, g1, o0, o1, i0, i1, i2, i3):
        c = lax.axis_index("c")
        s = lax.axis_index("s")
        wid = s * NC + c
        xrow0 = wid * nblk * xrows_per_blk
        row0 = wid * nblk * BLOCK
        gsem = (g0, g1)
        osem = (o0, o1)
        isem = (i0, i1, i2, i3)

        def idx_copy(islot, blk):
            off = xrow0 + blk * xrows_per_blk
            return pltpu.make_async_copy(
                x_hbm.at[pl.ds(off, xrows_per_blk)], idx_v.at[islot],
                isem[islot])

        def gather_copy(islot, slot, j):
            return pltpu.make_async_copy(
                table_hbm.at[idx_v.at[islot, j]],
                rows_v.at[slot, pl.ds(j * K, K)],
                gsem[slot])

        def out_copy(slot, blk):
            off = row0 + blk * BLOCK
            return pltpu.make_async_copy(
                rows_v.at[slot], out_hbm.at[pl.ds(off, BLOCK)], osem[slot])

        def fire_gathers(blk_slot4, blk_slot2):
            for j in range(CHUNKS):
                gather_copy(blk_slot4, blk_slot2, j).start()

        idx_copy(0, 0).start()
        idx_copy(1, 1).start()
        idx_copy(0, 0).wait()
        fire_gathers(0, 0)

        def body(it, carry):
            for p in range(4):
                g = it * 4 + p
                slot = p % 2
                islot = p

                @pl.when(g + 2 < nblk)
                def _():
                    idx_copy((p + 2) % 4, g + 2).start()

                for j in range(CHUNKS):
                    gather_copy(islot, slot, j).wait()
                out_copy(slot, g).start()

                @pl.when(g + 1 < nblk)
                def _():
                    @pl.when(g > 0)
                    def _():
                        out_copy(1 - slot, g - 1).wait()
                    idx_copy((p + 1) % 4, g + 1).wait()
                    fire_gathers((p + 1) % 4, 1 - slot)
            return carry

        lax.fori_loop(0, nblk // 4, body, 0)

        out_copy(0, nblk - 2).wait()
        out_copy(1, nblk - 1).wait()

    return k


def kernel(x, atom_emb_weight):
    B = x.shape[0] * x.shape[1]
    x2 = x.reshape(B // K, K).astype(jnp.int32)
    out = _make_kernel(B, atom_emb_weight.shape[0])(x2, atom_emb_weight)
    return out.reshape(x.shape + (EMB,))
